# baseline (speedup 1.0000x reference)
import functools

import jax
import jax.numpy as jnp
from jax import lax
from jax.experimental import pallas as pl
from jax.experimental.pallas import tpu as pltpu, tpu_sc as plsc

VOCAB = 100000
DIM = 128

N_CORES = 2
N_SUBCORES = 16
N_WORKERS = N_CORES * N_SUBCORES

N_SENT = 4096
SENT_LEN = 50
TOTAL = N_SENT * SENT_LEN
ROWS_PER_W = TOTAL // N_WORKERS
GCHUNK = 128
STEP = 256
GPS = STEP // GCHUNK
NSTEP = ROWS_PER_W // STEP
NBUF = 3
SKEW = 1

_MESH = plsc.VectorSubcoreMesh(core_axis_name="c", subcore_axis_name="s")


@functools.partial(
    pl.kernel,
    out_type=jax.ShapeDtypeStruct((TOTAL, DIM), jnp.float32),
    mesh=_MESH,
    scratch_types=[
        pltpu.VMEM((ROWS_PER_W,), jnp.int32),
        pltpu.VMEM((NBUF, STEP, DIM), jnp.float32),
        pltpu.SemaphoreType.DMA((NBUF,)),
        pltpu.SemaphoreType.DMA((NBUF,)),
    ],
)
def _gather_kernel(idx_hbm, table_hbm, out_hbm, idx_v, rows_v, gsem, ssem):
    wid = lax.axis_index("s") * N_CORES + lax.axis_index("c")
    base = wid * ROWS_PER_W
    pltpu.sync_copy(idx_hbm.at[pl.ds(base, ROWS_PER_W)], idx_v)

    def gathers(s):
        b = lax.rem(s, NBUF) if not isinstance(s, int) else s % NBUF
        return [
            pltpu.make_async_copy(
                table_hbm.at[idx_v.at[pl.ds(s * STEP + k * GCHUNK, GCHUNK)]],
                rows_v.at[b, pl.ds(k * GCHUNK, GCHUNK)],
                gsem.at[b])
            for k in range(GPS)
        ]

    def store(s):
        b = lax.rem(s, NBUF) if not isinstance(s, int) else s % NBUF
        return pltpu.make_async_copy(
            rows_v.at[b], out_hbm.at[pl.ds(base + s * STEP, STEP)],
            ssem.at[b])

    def start_gathers(s):
        for g in gathers(s):
            g.start()

    def wait_gathers(s):
        for g in gathers(s):
            g.wait()

    for i in range(SKEW):
        start_gathers(i)
    for i in range(SKEW, NBUF):
        wait_gathers(i - SKEW)
        store(i - SKEW).start()
        start_gathers(i)

    def steady(i, _):
        store(i - NBUF).wait()
        wait_gathers(i - SKEW)
        store(i - SKEW).start()
        start_gathers(i)
        return 0

    lax.fori_loop(NBUF, NSTEP, steady, 0, unroll=False)

    for i in range(NSTEP, NSTEP + SKEW):
        store(i - NBUF).wait()
        wait_gathers(i - SKEW)
        store(i - SKEW).start()
    for s in range(NSTEP - NBUF + SKEW, NSTEP):
        store(s).wait()


def kernel(indices, table):
    idx = indices.astype(jnp.int32).T.reshape(TOTAL)
    out = _gather_kernel(idx, table)
    return out.reshape(SENT_LEN, N_SENT, DIM).transpose(1, 0, 2)


# speedup vs baseline: 1.0492x; 1.0492x over previous
import functools

import jax
import jax.numpy as jnp
from jax import lax
from jax.experimental import pallas as pl
from jax.experimental.pallas import tpu as pltpu, tpu_sc as plsc

VOCAB = 100000
DIM = 128

N_CORES = 2
N_SUBCORES = 16
N_WORKERS = N_CORES * N_SUBCORES

N_SENT = 4096
SENT_LEN = 50
SENT_PER_W = N_SENT // N_WORKERS
NBUF = 6
SKEW = 3

_MESH = plsc.VectorSubcoreMesh(core_axis_name="c", subcore_axis_name="s")


@functools.partial(
    pl.kernel,
    out_type=jax.ShapeDtypeStruct((SENT_LEN, N_SENT, DIM), jnp.float32),
    mesh=_MESH,
    scratch_types=[
        pltpu.VMEM((SENT_LEN, SENT_PER_W), jnp.int32),
        pltpu.VMEM((NBUF, SENT_PER_W, DIM), jnp.float32),
        pltpu.SemaphoreType.DMA((NBUF,)),
        pltpu.SemaphoreType.DMA((NBUF,)),
    ],
)
def _gather_kernel(idx_hbm, table_hbm, out_hbm, idx_v, rows_v, gsem, ssem):
    wid = lax.axis_index("s") * N_CORES + lax.axis_index("c")
    sent_base = wid * SENT_PER_W
    pltpu.sync_copy(
        idx_hbm.at[pl.ds(0, SENT_LEN), pl.ds(sent_base, SENT_PER_W)], idx_v)

    def gather(j):
        b = lax.rem(j, NBUF) if not isinstance(j, int) else j % NBUF
        return pltpu.make_async_copy(
            table_hbm.at[idx_v.at[j]], rows_v.at[b], gsem.at[b])

    def store(j):
        b = lax.rem(j, NBUF) if not isinstance(j, int) else j % NBUF
        return pltpu.make_async_copy(
            rows_v.at[b], out_hbm.at[j, pl.ds(sent_base, SENT_PER_W)],
            ssem.at[b])

    for i in range(SKEW):
        gather(i).start()
    for i in range(SKEW, NBUF):
        gather(i - SKEW).wait()
        store(i - SKEW).start()
        gather(i).start()

    def steady(i, _):
        store(i - NBUF).wait()
        gather(i - SKEW).wait()
        store(i - SKEW).start()
        gather(i).start()
        return 0

    lax.fori_loop(NBUF, SENT_LEN, steady, 0, unroll=False)

    for i in range(SENT_LEN, SENT_LEN + SKEW):
        store(i - NBUF).wait()
        gather(i - SKEW).wait()
        store(i - SKEW).start()
    for j in range(SENT_LEN - NBUF + SKEW, SENT_LEN):
        store(j).wait()


def kernel(indices, table):
    idx = indices.astype(jnp.int32).T
    out = _gather_kernel(idx, table)
    return out.transpose(1, 0, 2)


# speedup vs baseline: 1.0504x; 1.0011x over previous
import functools

import jax
import jax.numpy as jnp
from jax import lax
from jax.experimental import pallas as pl
from jax.experimental.pallas import tpu as pltpu, tpu_sc as plsc

VOCAB = 100000
DIM = 128

N_CORES = 2
N_SUBCORES = 16
N_WORKERS = N_CORES * N_SUBCORES

N_SENT = 4096
SENT_LEN = 50
SENT_PER_W = N_SENT // N_WORKERS
NBUF = 6
SKEW = 3

_MESH = plsc.VectorSubcoreMesh(core_axis_name="c", subcore_axis_name="s")


@functools.partial(
    pl.kernel,
    out_type=jax.ShapeDtypeStruct((SENT_LEN, N_SENT, DIM), jnp.float32),
    mesh=_MESH,
    scratch_types=[
        pltpu.VMEM((SENT_LEN, SENT_PER_W), jnp.int32),
        pltpu.VMEM((NBUF, SENT_PER_W, DIM), jnp.float32),
        pltpu.SemaphoreType.DMA((NBUF,)),
        pltpu.SemaphoreType.DMA((NBUF,)),
        pltpu.SemaphoreType.DMA,
    ],
)
def _gather_kernel(idx_hbm, table_hbm, out_hbm, idx_v, rows_v, gsem, ssem,
                   isem):
    wid = lax.axis_index("s") * N_CORES + lax.axis_index("c")
    sent_base = wid * SENT_PER_W
    IDX_HEAD = 8
    pltpu.sync_copy(
        idx_hbm.at[pl.ds(0, IDX_HEAD), pl.ds(sent_base, SENT_PER_W)],
        idx_v.at[pl.ds(0, IDX_HEAD)])
    idx_rest = pltpu.make_async_copy(
        idx_hbm.at[pl.ds(IDX_HEAD, SENT_LEN - IDX_HEAD),
                   pl.ds(sent_base, SENT_PER_W)],
        idx_v.at[pl.ds(IDX_HEAD, SENT_LEN - IDX_HEAD)], isem)
    idx_rest.start()

    def gather(j):
        b = lax.rem(j, NBUF) if not isinstance(j, int) else j % NBUF
        return pltpu.make_async_copy(
            table_hbm.at[idx_v.at[j]], rows_v.at[b], gsem.at[b])

    def store(j):
        b = lax.rem(j, NBUF) if not isinstance(j, int) else j % NBUF
        return pltpu.make_async_copy(
            rows_v.at[b], out_hbm.at[j, pl.ds(sent_base, SENT_PER_W)],
            ssem.at[b])

    for i in range(SKEW):
        gather(i).start()
    for i in range(SKEW, NBUF):
        gather(i - SKEW).wait()
        store(i - SKEW).start()
        gather(i).start()

    idx_rest.wait()

    def steady(i, _):
        store(i - NBUF).wait()
        gather(i - SKEW).wait()
        store(i - SKEW).start()
        gather(i).start()
        return 0

    lax.fori_loop(NBUF, SENT_LEN, steady, 0, unroll=False)

    for i in range(SENT_LEN, SENT_LEN + SKEW):
        store(i - NBUF).wait()
        gather(i - SKEW).wait()
        store(i - SKEW).start()
    for j in range(SENT_LEN - NBUF + SKEW, SENT_LEN):
        store(j).wait()


def kernel(indices, table):
    idx = indices.astype(jnp.int32).T
    out = _gather_kernel(idx, table)
    return out.transpose(1, 0, 2)


# speedup vs baseline: 1.0504x; 1.0000x over previous
import functools

import jax
import jax.numpy as jnp
from jax import lax
from jax.experimental import pallas as pl
from jax.experimental.pallas import tpu as pltpu, tpu_sc as plsc

VOCAB = 100000
DIM = 128

N_CORES = 2
N_SUBCORES = 16
N_WORKERS = N_CORES * N_SUBCORES

N_SENT = 4096
SENT_LEN = 50
SENT_PER_W = N_SENT // N_WORKERS
NBUF = 6
SKEW = 3

_MESH = plsc.VectorSubcoreMesh(core_axis_name="c", subcore_axis_name="s")


@functools.partial(
    pl.kernel,
    out_type=jax.ShapeDtypeStruct((SENT_LEN, N_SENT, DIM), jnp.float32),
    mesh=_MESH,
    compiler_params=pltpu.CompilerParams(skip_device_barrier=True),
    scratch_types=[
        pltpu.VMEM((SENT_LEN, SENT_PER_W), jnp.int32),
        pltpu.VMEM((NBUF, SENT_PER_W, DIM), jnp.float32),
        pltpu.SemaphoreType.DMA((NBUF,)),
        pltpu.SemaphoreType.DMA((NBUF,)),
        pltpu.SemaphoreType.DMA,
    ],
)
def _gather_kernel(idx_hbm, table_hbm, out_hbm, idx_v, rows_v, gsem, ssem,
                   isem):
    wid = lax.axis_index("s") * N_CORES + lax.axis_index("c")
    sent_base = wid * SENT_PER_W
    IDX_HEAD = 8
    pltpu.sync_copy(
        idx_hbm.at[pl.ds(0, IDX_HEAD), pl.ds(sent_base, SENT_PER_W)],
        idx_v.at[pl.ds(0, IDX_HEAD)])
    idx_rest = pltpu.make_async_copy(
        idx_hbm.at[pl.ds(IDX_HEAD, SENT_LEN - IDX_HEAD),
                   pl.ds(sent_base, SENT_PER_W)],
        idx_v.at[pl.ds(IDX_HEAD, SENT_LEN - IDX_HEAD)], isem)
    idx_rest.start()

    def gather(j):
        b = lax.rem(j, NBUF) if not isinstance(j, int) else j % NBUF
        return pltpu.make_async_copy(
            table_hbm.at[idx_v.at[j]], rows_v.at[b], gsem.at[b])

    def store(j):
        b = lax.rem(j, NBUF) if not isinstance(j, int) else j % NBUF
        return pltpu.make_async_copy(
            rows_v.at[b], out_hbm.at[j, pl.ds(sent_base, SENT_PER_W)],
            ssem.at[b])

    for i in range(SKEW):
        gather(i).start()
    for i in range(SKEW, NBUF):
        gather(i - SKEW).wait()
        store(i - SKEW).start()
        gather(i).start()

    idx_rest.wait()

    def steady(i, _):
        store(i - NBUF).wait()
        gather(i - SKEW).wait()
        store(i - SKEW).start()
        gather(i).start()
        return 0

    lax.fori_loop(NBUF, SENT_LEN, steady, 0, unroll=False)

    for i in range(SENT_LEN, SENT_LEN + SKEW):
        store(i - NBUF).wait()
        gather(i - SKEW).wait()
        store(i - SKEW).start()
    for j in range(SENT_LEN - NBUF + SKEW, SENT_LEN):
        store(j).wait()


def kernel(indices, table):
    idx = indices.astype(jnp.int32).T
    out = _gather_kernel(idx, table)
    return out.transpose(1, 0, 2)


# speedup vs baseline: 1.0521x; 1.0016x over previous
import functools

import jax
import jax.numpy as jnp
from jax import lax
from jax.experimental import pallas as pl
from jax.experimental.pallas import tpu as pltpu, tpu_sc as plsc

VOCAB = 100000
DIM = 128

N_CORES = 2
N_SUBCORES = 16
N_WORKERS = N_CORES * N_SUBCORES

N_SENT = 4096
SENT_LEN = 50
SENT_PER_W = N_SENT // N_WORKERS
NBUF = 6
SKEW = 3

_MESH = plsc.VectorSubcoreMesh(core_axis_name="c", subcore_axis_name="s")


@functools.partial(
    pl.kernel,
    out_type=jax.ShapeDtypeStruct((SENT_LEN, N_SENT, DIM), jnp.float32),
    mesh=_MESH,
    scratch_types=[
        pltpu.VMEM((SENT_LEN, SENT_PER_W), jnp.int32),
        pltpu.VMEM((NBUF, SENT_PER_W, DIM), jnp.float32),
        pltpu.SemaphoreType.DMA((NBUF,)),
        pltpu.SemaphoreType.DMA((NBUF,)),
        pltpu.SemaphoreType.DMA,
    ],
)
def _gather_kernel(idx_hbm, table_hbm, out_hbm, idx_v, rows_v, gsem, ssem,
                   isem):
    wid = lax.axis_index("s") * N_CORES + lax.axis_index("c")
    sent_base = wid * SENT_PER_W
    IDX_HEAD = 8
    pltpu.sync_copy(
        idx_hbm.at[pl.ds(0, IDX_HEAD), pl.ds(sent_base, SENT_PER_W)],
        idx_v.at[pl.ds(0, IDX_HEAD)])
    idx_rest = pltpu.make_async_copy(
        idx_hbm.at[pl.ds(IDX_HEAD, SENT_LEN - IDX_HEAD),
                   pl.ds(sent_base, SENT_PER_W)],
        idx_v.at[pl.ds(IDX_HEAD, SENT_LEN - IDX_HEAD)], isem)
    idx_rest.start()

    def gather(j):
        b = lax.rem(j, NBUF) if not isinstance(j, int) else j % NBUF
        return pltpu.make_async_copy(
            table_hbm.at[idx_v.at[j]], rows_v.at[b], gsem.at[b])

    def store(j):
        b = lax.rem(j, NBUF) if not isinstance(j, int) else j % NBUF
        return pltpu.make_async_copy(
            rows_v.at[b], out_hbm.at[j, pl.ds(sent_base, SENT_PER_W)],
            ssem.at[b])

    for i in range(SKEW):
        gather(i).start()
    for i in range(SKEW, NBUF):
        gather(i - SKEW).wait()
        store(i - SKEW).start()
        gather(i).start()

    idx_rest.wait()

    def steady(i, _):
        store(i - NBUF).wait()
        gather(i - SKEW).wait()
        store(i - SKEW).start()
        gather(i).start()
        return 0

    lax.fori_loop(NBUF, SENT_LEN, steady, 0, unroll=False)

    for i in range(SENT_LEN, SENT_LEN + SKEW):
        store(i - NBUF).wait()
        gather(i - SKEW).wait()
        store(i - SKEW).start()
    for j in range(SENT_LEN - NBUF + SKEW, SENT_LEN):
        store(j).wait()


def kernel(indices, table):
    idx = indices.astype(jnp.int32).T
    out = _gather_kernel(idx, table)
    return out.transpose(1, 0, 2)
